# baseline (speedup 1.0000x reference)
import functools
import math

import jax
import jax.numpy as jnp
from jax import lax
from jax.experimental import pallas as pl
from jax.experimental.pallas import tpu as pltpu
from jax.experimental.pallas import tpu_sc as plsc

PAD = 0
RATE = 0.1
NC = 2
NS = 16
LANES = 16
RPW = 64
VT = 22400
CW = 3200
NBUF = 2
UNROLL = 40
GW = 32


def _tc_total_body(h_ref, o_ref):
    i = pl.program_id(0)
    j = pl.program_id(1)

    @pl.when(jnp.logical_and(i == 0, j == 0))
    def _init():
        o_ref[...] = jnp.zeros_like(o_ref)

    o_ref[...] += jnp.sum(h_ref[...]).reshape(1, 1)


def _sc_body(hyp_hbm, refs_hbm, out_hbm, refs_v, tiles_v, h0tiles_v, sbuf_v,
             res_v, gsem, hsem, *csems, s, v, c, tail0):
    wid = lax.axis_index("s") * NC + lax.axis_index("c")
    base = wid * RPW
    b = base // s
    s0 = base % s
    iota = lax.iota(jnp.int32, LANES)
    pltpu.sync_copy(refs_hbm.at[pl.ds(base, RPW)], refs_v)

    def rscalar(k):
        chunk = refs_v[pl.ds((k // LANES) * LANES, LANES)]
        return chunk[k % LANES]

    def issue_gold_wave(wave):
        cps = []
        for kk in range(GW):
            k = wave * GW + kk
            r = rscalar(k)
            v128 = pl.multiple_of((r >> 7) << 7, 128)
            s8 = ((s0 + k) // 8) * 8
            cps.append(pltpu.async_copy(
                hyp_hbm.at[b, pl.ds(s8, 8), pl.ds(v128, 128)],
                tiles_v.at[kk], gsem))
        return cps

    gather_copies = issue_gold_wave(0)
    for g in range(RPW // 8):
        s8 = ((s0 + g * 8) // 8) * 8
        gather_copies.append(pltpu.async_copy(
            hyp_hbm.at[b, pl.ds(s8, 8), pl.ds(0, 128)], h0tiles_v.at[g], hsem))

    w = v - VT
    cpb = w // CW
    nch = (RPW // 8) * cpb

    def chunk_src(cidx):
        band = cidx // cpb
        off = VT + (cidx % cpb) * CW
        s8 = pl.multiple_of(s0 + band * 8, 8)
        return hyp_hbm.at[b, pl.ds(s8, 8), pl.ds(pl.multiple_of(off, 128), CW)]

    for j in range(NBUF):
        pltpu.async_copy(chunk_src(j), sbuf_v.at[j], csems[j])

    def q_body(q, acc):
        for j in range(NBUF):
            cidx = q * NBUF + j
            pltpu.make_async_copy(
                hyp_hbm.at[b, pl.ds(0, 8), pl.ds(0, CW)], sbuf_v.at[j],
                csems[j]).wait()
            for row in range(8):
                def sum_body(i, a):
                    off = i * (LANES * UNROLL)
                    vals = [
                        sbuf_v[j, row,
                               pl.ds(pl.multiple_of(off + u * LANES, LANES),
                                     LANES)]
                        for u in range(UNROLL)
                    ]
                    while len(vals) > 1:
                        nxt = [vals[p] + vals[p + 1]
                               for p in range(0, len(vals) - 1, 2)]
                        if len(vals) % 2:
                            nxt.append(vals[-1])
                        vals = nxt
                    return a + vals[0]
                acc = lax.fori_loop(0, CW // (LANES * UNROLL), sum_body, acc)

            @pl.when(cidx + NBUF < nch)
            def _prefetch():
                pltpu.async_copy(chunk_src(cidx + NBUF), sbuf_v.at[j], csems[j])
        return acc

    acc = lax.fori_loop(0, nch // NBUF, q_body,
                        jnp.zeros((LANES,), jnp.float32))

    gcoef = (RATE - c) / c
    mlogc = -math.log(c)
    mlogc_vec = jnp.full((LANES,), mlogc, jnp.float32)
    for wave in range(RPW // GW):
        for cp in gather_copies:
            cp.wait()
        for kk in range(GW):
            k = wave * GW + kk
            r = rscalar(k)
            srow = (s0 + k) % 8
            off16 = pl.multiple_of(((r & 127) >> 4) << 4, 16)
            chunk = tiles_v[kk, srow, pl.ds(off16, 16)]
            acc = acc + jnp.where(iota == (r & 15), gcoef * chunk, 0.0)
            h0c = h0tiles_v[k // 8, srow, pl.ds(0, 16)]
            t0 = jnp.where(r != PAD, 1.0, 0.0)
            acc = acc + jnp.where(iota == 0,
                                  (-t0) * h0c + (1.0 - t0) * mlogc_vec, 0.0)
        if wave + 1 < RPW // GW:
            gather_copies = issue_gold_wave(wave + 1)
    res_v[...] = acc
    pltpu.sync_copy(res_v, out_hbm.at[wid])


def kernel(hypotheses, references):
    B, S, V = hypotheses.shape
    N = B * S
    c = (1.0 - RATE) / (V - 2)
    k1 = (V - 2) * c * math.log(c) + RATE * math.log(RATE)
    nw = NC * NS
    VB = 3200
    NVT = VT // VB
    refs_flat = references.astype(jnp.int32).reshape(N)

    total_tc = pl.pallas_call(
        _tc_total_body,
        grid=(B, NVT),
        in_specs=[pl.BlockSpec((1, S, VB), lambda i, j: (i, 0, j))],
        out_specs=pl.BlockSpec((1, 1), lambda i, j: (0, 0)),
        out_shape=jax.ShapeDtypeStruct((1, 1), jnp.float32),
    )(hypotheses)[0, 0]

    mesh = plsc.VectorSubcoreMesh(core_axis_name="c", subcore_axis_name="s")
    sc_partials = pl.kernel(
        functools.partial(_sc_body, s=S, v=V, c=c, tail0=c * math.log(c)),
        out_type=jax.ShapeDtypeStruct((nw, LANES), jnp.float32),
        mesh=mesh,
        scratch_types=[
            pltpu.VMEM((RPW,), jnp.int32),
            pltpu.VMEM((GW, 8, 128), jnp.float32),
            pltpu.VMEM((RPW // 8, 8, 128), jnp.float32),
            pltpu.VMEM((NBUF, 8, CW), jnp.float32),
            pltpu.VMEM((LANES,), jnp.float32),
            pltpu.SemaphoreType.DMA,
            pltpu.SemaphoreType.DMA,
        ] + [pltpu.SemaphoreType.DMA] * NBUF,
        compiler_params=pltpu.CompilerParams(use_tc_tiling_on_sc=True),
    )(hypotheses, refs_flat)

    return N * k1 - c * (total_tc + jnp.sum(sc_partials))


# speedup vs baseline: 1.3066x; 1.3066x over previous
import functools
import math

import jax
import jax.numpy as jnp
from jax import lax
from jax.experimental import pallas as pl
from jax.experimental.pallas import tpu as pltpu
from jax.experimental.pallas import tpu_sc as plsc

PAD = 0
RATE = 0.1
NC = 2
NS = 16
LANES = 16
RPW = 64
VT = 19200
CW = 3200
NBUF = 2
UNROLL = 40
GW = 32


def _tc_total_body(h_ref, o_ref):
    i = pl.program_id(0)
    j = pl.program_id(1)

    @pl.when(jnp.logical_and(i == 0, j == 0))
    def _init():
        o_ref[...] = jnp.zeros_like(o_ref)

    o_ref[...] += jnp.sum(h_ref[...]).reshape(1, 1)


def _sc_body(hyp_hbm, refs_hbm, out_hbm, refs_v, tiles_v, h0tiles_v, sbuf_v,
             res_v, gsem, hsem, *csems, s, v, c, tail0):
    wid = lax.axis_index("s") * NC + lax.axis_index("c")
    base = wid * RPW
    b = base // s
    s0 = base % s
    iota = lax.iota(jnp.int32, LANES)
    pltpu.sync_copy(refs_hbm.at[pl.ds(base, RPW)], refs_v)

    def rscalar(k):
        chunk = refs_v[pl.ds((k // LANES) * LANES, LANES)]
        return chunk[k % LANES]

    def issue_gold_wave(wave):
        cps = []
        for kk in range(GW):
            k = wave * GW + kk
            r = rscalar(k)
            v128 = pl.multiple_of((r >> 7) << 7, 128)
            s8 = ((s0 + k) // 8) * 8
            cps.append(pltpu.async_copy(
                hyp_hbm.at[b, pl.ds(s8, 8), pl.ds(v128, 128)],
                tiles_v.at[kk], gsem))
        return cps

    gather_copies = issue_gold_wave(0)
    for g in range(RPW // 8):
        s8 = ((s0 + g * 8) // 8) * 8
        gather_copies.append(pltpu.async_copy(
            hyp_hbm.at[b, pl.ds(s8, 8), pl.ds(0, 128)], h0tiles_v.at[g], hsem))

    w = v - VT
    cpb = w // CW
    nch = (RPW // 8) * cpb

    def chunk_src(cidx):
        band = cidx // cpb
        off = VT + (cidx % cpb) * CW
        s8 = pl.multiple_of(s0 + band * 8, 8)
        return hyp_hbm.at[b, pl.ds(s8, 8), pl.ds(pl.multiple_of(off, 128), CW)]

    for j in range(NBUF):
        pltpu.async_copy(chunk_src(j), sbuf_v.at[j], csems[j])

    def q_body(q, acc):
        for j in range(NBUF):
            cidx = q * NBUF + j
            pltpu.make_async_copy(
                hyp_hbm.at[b, pl.ds(0, 8), pl.ds(0, CW)], sbuf_v.at[j],
                csems[j]).wait()
            for row in range(8):
                def sum_body(i, a):
                    off = i * (LANES * UNROLL)
                    vals = [
                        sbuf_v[j, row,
                               pl.ds(pl.multiple_of(off + u * LANES, LANES),
                                     LANES)]
                        for u in range(UNROLL)
                    ]
                    while len(vals) > 1:
                        nxt = [vals[p] + vals[p + 1]
                               for p in range(0, len(vals) - 1, 2)]
                        if len(vals) % 2:
                            nxt.append(vals[-1])
                        vals = nxt
                    return a + vals[0]
                acc = lax.fori_loop(0, CW // (LANES * UNROLL), sum_body, acc)

            @pl.when(cidx + NBUF < nch)
            def _prefetch():
                pltpu.async_copy(chunk_src(cidx + NBUF), sbuf_v.at[j], csems[j])
        return acc

    acc = lax.fori_loop(0, nch // NBUF, q_body,
                        jnp.zeros((LANES,), jnp.float32))

    gcoef = (RATE - c) / c
    mlogc = -math.log(c)
    mlogc_vec = jnp.full((LANES,), mlogc, jnp.float32)
    for wave in range(RPW // GW):
        for cp in gather_copies:
            cp.wait()
        for kk in range(GW):
            k = wave * GW + kk
            r = rscalar(k)
            srow = (s0 + k) % 8
            off16 = pl.multiple_of(((r & 127) >> 4) << 4, 16)
            chunk = tiles_v[kk, srow, pl.ds(off16, 16)]
            acc = acc + jnp.where(iota == (r & 15), gcoef * chunk, 0.0)
            h0c = h0tiles_v[k // 8, srow, pl.ds(0, 16)]
            t0 = jnp.where(r != PAD, 1.0, 0.0)
            acc = acc + jnp.where(iota == 0,
                                  (-t0) * h0c + (1.0 - t0) * mlogc_vec, 0.0)
        if wave + 1 < RPW // GW:
            gather_copies = issue_gold_wave(wave + 1)
    res_v[...] = acc
    pltpu.sync_copy(res_v, out_hbm.at[wid])


def kernel(hypotheses, references):
    B, S, V = hypotheses.shape
    N = B * S
    c = (1.0 - RATE) / (V - 2)
    k1 = (V - 2) * c * math.log(c) + RATE * math.log(RATE)
    nw = NC * NS
    VB = 6400
    NVT = VT // VB
    refs_flat = references.astype(jnp.int32).reshape(N)

    total_tc = pl.pallas_call(
        _tc_total_body,
        grid=(B, NVT),
        in_specs=[pl.BlockSpec((1, S, VB), lambda i, j: (i, 0, j))],
        out_specs=pl.BlockSpec((1, 1), lambda i, j: (0, 0)),
        out_shape=jax.ShapeDtypeStruct((1, 1), jnp.float32),
    )(hypotheses)[0, 0]

    mesh = plsc.VectorSubcoreMesh(core_axis_name="c", subcore_axis_name="s")
    sc_partials = pl.kernel(
        functools.partial(_sc_body, s=S, v=V, c=c, tail0=c * math.log(c)),
        out_type=jax.ShapeDtypeStruct((nw, LANES), jnp.float32),
        mesh=mesh,
        scratch_types=[
            pltpu.VMEM((RPW,), jnp.int32),
            pltpu.VMEM((GW, 8, 128), jnp.float32),
            pltpu.VMEM((RPW // 8, 8, 128), jnp.float32),
            pltpu.VMEM((NBUF, 8, CW), jnp.float32),
            pltpu.VMEM((LANES,), jnp.float32),
            pltpu.SemaphoreType.DMA,
            pltpu.SemaphoreType.DMA,
        ] + [pltpu.SemaphoreType.DMA] * NBUF,
        compiler_params=pltpu.CompilerParams(use_tc_tiling_on_sc=True),
    )(hypotheses, refs_flat)

    return N * k1 - c * (total_tc + jnp.sum(sc_partials))
